# Initial kernel scaffold; baseline (speedup 1.0000x reference)
import jax
import jax.numpy as jnp
from jax.experimental import pallas as pl


def kernel(nh, eh, edge_index, nf_params, ef_params):
    raise NotImplementedError("write your pallas kernel here")


# speedup vs baseline: 5.2907x; 5.2907x over previous
import functools

import jax
import jax.numpy as jnp
from jax import lax
from jax.experimental import pallas as pl
from jax.experimental.pallas import tpu as pltpu
from jax.experimental.pallas import tpu_sc as plsc

NN = 10000
NE = 320000
D = 128
NCORE = 2
NSUB = 16
NWORK = NCORE * NSUB
PERW = NE // NWORK
CH = 128
NFULL = PERW // CH
TAIL = PERW - NFULL * CH


def _mesh():
    return plsc.VectorSubcoreMesh(core_axis_name="c", subcore_axis_name="s",
                                  num_cores=NCORE, num_subcores=NSUB)


def _mlp_body(x_ref, w_ref, b_ref, o_ref):
    h = x_ref[...]
    for l in range(4):
        h = lax.dot_general(h, w_ref[l], (((1,), (0,)), ((), ())),
                            preferred_element_type=jnp.float32)
        h = jnp.tanh(h + b_ref[l][None, :])
    o_ref[...] = h


def _mlp(x, ws, bs, blk):
    n = x.shape[0]
    return pl.pallas_call(
        _mlp_body,
        grid=(n // blk,),
        in_specs=[pl.BlockSpec((blk, D), lambda i: (i, 0)),
                  pl.BlockSpec((4, D, D), lambda i: (0, 0, 0)),
                  pl.BlockSpec((4, D), lambda i: (0, 0))],
        out_specs=pl.BlockSpec((blk, D), lambda i: (i, 0)),
        out_shape=jax.ShapeDtypeStruct((n, D), jnp.float32),
    )(x, ws, bs)


EBLK = 2560


def _attn_body(gs_ref, gd_ref, eh_ref, attn_ref, m_ref):
    i = pl.program_id(0)
    a = jnp.sum(gs_ref[...] * eh_ref[...] * gd_ref[...], axis=1, keepdims=True)
    attn_ref[...] = a
    bm = jnp.max(a)

    @pl.when(i == 0)
    def _():
        m_ref[0, 0] = bm

    @pl.when(i > 0)
    def _():
        m_ref[0, 0] = jnp.maximum(m_ref[0, 0], bm)


def _attn(gsrc, gdst, eh):
    return pl.pallas_call(
        _attn_body,
        grid=(NE // EBLK,),
        in_specs=[pl.BlockSpec((EBLK, D), lambda i: (i, 0)),
                  pl.BlockSpec((EBLK, D), lambda i: (i, 0)),
                  pl.BlockSpec((EBLK, D), lambda i: (i, 0))],
        out_specs=[pl.BlockSpec((EBLK, 1), lambda i: (i, 0)),
                   pl.BlockSpec((1, 1), lambda i: (0, 0),
                                memory_space=pltpu.SMEM)],
        out_shape=[jax.ShapeDtypeStruct((NE, 1), jnp.float32),
                   jax.ShapeDtypeStruct((1, 1), jnp.float32)],
    )(gsrc, gdst, eh)


def _pw_body(attn_ref, m_ref, gs_ref, w_ref, p_ref):
    p = jnp.exp(attn_ref[...] - m_ref[0, 0])
    w_ref[...] = gs_ref[...] * p
    p_ref[...] = p


def _pw(attn, m, gsrc):
    return pl.pallas_call(
        _pw_body,
        grid=(NE // EBLK,),
        in_specs=[pl.BlockSpec((EBLK, 1), lambda i: (i, 0)),
                  pl.BlockSpec((1, 1), lambda i: (0, 0),
                               memory_space=pltpu.SMEM),
                  pl.BlockSpec((EBLK, D), lambda i: (i, 0))],
        out_specs=[pl.BlockSpec((EBLK, D), lambda i: (i, 0)),
                   pl.BlockSpec((EBLK, 1), lambda i: (i, 0))],
        out_shape=[jax.ShapeDtypeStruct((NE, D), jnp.float32),
                   jax.ShapeDtypeStruct((NE, 1), jnp.float32)],
    )(attn, m, gsrc)


def _sc_gather(table, idx):
    d = table.shape[1]

    @functools.partial(
        pl.kernel, mesh=_mesh(),
        out_type=jax.ShapeDtypeStruct((idx.shape[0], d), jnp.float32),
        scratch_types=[pltpu.VMEM((CH,), jnp.int32),
                       pltpu.VMEM((CH, d), jnp.float32),
                       pltpu.SemaphoreType.DMA],
    )
    def k(table_hbm, idx_hbm, out_hbm, idx_v, rows_v, sem):
        wid = lax.axis_index("s") * NCORE + lax.axis_index("c")
        base = wid * PERW

        def chunk(off, size):
            idx_b = idx_v.at[pl.ds(0, size)]
            rows_b = rows_v.at[pl.ds(0, size)]
            pltpu.sync_copy(idx_hbm.at[pl.ds(off, size)], idx_b)
            pltpu.async_copy(table_hbm.at[idx_b], rows_b, sem).wait()
            pltpu.sync_copy(rows_b, out_hbm.at[pl.ds(off, size)])

        @pl.loop(0, NFULL)
        def _(c):
            chunk(base + c * CH, CH)

        if TAIL:
            chunk(base + NFULL * CH, TAIL)

    return k(table, idx)


def _sc_scatter(w, p, dstidx, zeros):
    zr0 = 632
    zr_last = NN - 15 * zr0

    @functools.partial(
        pl.kernel, mesh=_mesh(),
        out_type=[jax.ShapeDtypeStruct((NCORE, NN, D), jnp.float32),
                  jax.ShapeDtypeStruct((NWORK, NN), jnp.float32)],
        scratch_types=[pltpu.VMEM((CH,), jnp.int32),
                       pltpu.VMEM((TAIL,), jnp.int32),
                       pltpu.VMEM((CH, D), jnp.float32),
                       pltpu.VMEM((CH,), jnp.float32),
                       pltpu.VMEM((TAIL,), jnp.float32),
                       pltpu.VMEM((NN,), jnp.float32),
                       pltpu.VMEM_SHARED((NN, D), jnp.float32)],
        compiler_params=pltpu.CompilerParams(needs_layout_passes=False),
    )
    def k(w_hbm, p_hbm, dst_hbm, z_hbm, out_hbm, s_hbm,
          idx_v, idx_t, rows_v, p_v, p_t, s_tile, acc_sh):
        c = lax.axis_index("c")
        s = lax.axis_index("s")

        def rows_slice(src_ref, dst_ref):
            @pl.when(s < 15)
            def _():
                pltpu.sync_copy(src_ref.at[pl.ds(s * zr0, zr0)],
                                dst_ref.at[pl.ds(s * zr0, zr0)])

            @pl.when(s == 15)
            def _():
                pltpu.sync_copy(src_ref.at[pl.ds(15 * zr0, zr_last)],
                                dst_ref.at[pl.ds(15 * zr0, zr_last)])

        rows_slice(z_hbm, acc_sh)
        zvec = jnp.zeros((16,), jnp.float32)

        @pl.loop(0, NN, step=16)
        def _(i):
            s_tile[pl.ds(i, 16)] = zvec

        plsc.subcore_barrier()

        base = (c * NSUB + s) * PERW

        def chunk(off, size, idx_b, p_b):
            rows_b = rows_v.at[pl.ds(0, size)]
            pltpu.sync_copy(dst_hbm.at[pl.ds(off, size)], idx_b)
            pltpu.sync_copy(w_hbm.at[pl.ds(off, size)], rows_b)
            pltpu.sync_copy(p_hbm.at[pl.ds(off, size)], p_b)
            pltpu.sync_copy(rows_b, acc_sh.at[idx_b], add=True)
            for g in range(size // 16):
                ii = idx_b[pl.ds(g * 16, 16)]
                pi = p_b[pl.ds(g * 16, 16)]
                plsc.addupdate_scatter(s_tile, [ii], pi)

        @pl.loop(0, NFULL)
        def _(ci):
            chunk(base + ci * CH, CH, idx_v, p_v)

        if TAIL:
            chunk(base + NFULL * CH, TAIL, idx_t, p_t)

        plsc.subcore_barrier()
        rows_slice(acc_sh, out_hbm.at[c])
        pltpu.sync_copy(s_tile, s_hbm.at[c * NSUB + s])

    return k(w, p, dstidx, zeros)


NBLK = 2000


def _scomb_body(s_ref, o_ref):
    o_ref[...] = lax.dot_general(s_ref[...], jnp.ones((NWORK, 1), jnp.float32),
                                 (((0,), (0,)), ((), ())),
                                 preferred_element_type=jnp.float32)


def _scomb(s_part):
    return pl.pallas_call(
        _scomb_body,
        out_shape=jax.ShapeDtypeStruct((NN, 1), jnp.float32),
    )(s_part)


def _fin_body(acc_ref, s_ref, nh_ref, nhout_ref, nz_ref):
    a = acc_ref[0] + acc_ref[1]
    s = s_ref[...]
    nz = jnp.where(s > 0.0, a / s, 0.0)
    nz_ref[...] = nz
    nhout_ref[...] = nh_ref[...] + nz


def _finalize(acc, s_col, n_h):
    return pl.pallas_call(
        _fin_body,
        grid=(NN // NBLK,),
        in_specs=[pl.BlockSpec((NCORE, NBLK, D), lambda i: (0, i, 0)),
                  pl.BlockSpec((NBLK, 1), lambda i: (i, 0)),
                  pl.BlockSpec((NBLK, D), lambda i: (i, 0))],
        out_specs=[pl.BlockSpec((NBLK, D), lambda i: (i, 0)),
                   pl.BlockSpec((NBLK, D), lambda i: (i, 0))],
        out_shape=[jax.ShapeDtypeStruct((NN, D), jnp.float32),
                   jax.ShapeDtypeStruct((NN, D), jnp.float32)],
    )(acc, s_col, n_h)


def _eadd_body(eh_ref, gnz_ref, o_ref):
    o_ref[...] = eh_ref[...] + gnz_ref[...]


def _eadd(e_h, gnz):
    return pl.pallas_call(
        _eadd_body,
        grid=(NE // EBLK,),
        in_specs=[pl.BlockSpec((EBLK, D), lambda i: (i, 0)),
                  pl.BlockSpec((EBLK, D), lambda i: (i, 0))],
        out_specs=pl.BlockSpec((EBLK, D), lambda i: (i, 0)),
        out_shape=jax.ShapeDtypeStruct((NE, D), jnp.float32),
    )(e_h, gnz)


def kernel(nh, eh, edge_index, nf_params, ef_params):
    src, dst = edge_index[0], edge_index[1]
    wn = jnp.stack([w for w, _ in nf_params])
    bn = jnp.stack([b for _, b in nf_params])
    we = jnp.stack([w for w, _ in ef_params])
    be = jnp.stack([b for _, b in ef_params])

    n_h = _mlp(nh, wn, bn, blk=2000)
    e_h = _mlp(eh, we, be, blk=EBLK)

    gsrc = _sc_gather(n_h, src)
    gdst = _sc_gather(n_h, dst)

    attn, m = _attn(gsrc, gdst, e_h)
    w128, p = _pw(attn, m, gsrc)

    zeros = jnp.zeros((NN, D), jnp.float32)
    acc, s_part = _sc_scatter(w128, p.reshape(NE), dst, zeros)

    nh_out, nz = _finalize(acc, _scomb(s_part), n_h)

    gnz = _sc_gather(nz, src)
    eh_out = _eadd(e_h, gnz)
    return (nh_out, eh_out)


# speedup vs baseline: 6.9061x; 1.3053x over previous
import functools

import jax
import jax.numpy as jnp
from jax import lax
from jax.experimental import pallas as pl
from jax.experimental.pallas import tpu as pltpu
from jax.experimental.pallas import tpu_sc as plsc

NN = 10000
NE = 320000
D = 128
NCORE = 2
NSUB = 16
NWORK = NCORE * NSUB
PERW = NE // NWORK
CH = 128
NFULL = PERW // CH
TAIL = PERW - NFULL * CH
SCH = 80
SNF = PERW // SCH


def _mesh():
    return plsc.VectorSubcoreMesh(core_axis_name="c", subcore_axis_name="s",
                                  num_cores=NCORE, num_subcores=NSUB)


def _mlp_body(x_ref, w_ref, b_ref, o_ref):
    h = x_ref[...]
    for l in range(4):
        h = lax.dot_general(h, w_ref[l], (((1,), (0,)), ((), ())),
                            preferred_element_type=jnp.float32)
        h = jnp.tanh(h + b_ref[l][None, :])
    o_ref[...] = h


def _mlp(x, ws, bs, blk):
    n = x.shape[0]
    return pl.pallas_call(
        _mlp_body,
        grid=(n // blk,),
        in_specs=[pl.BlockSpec((blk, D), lambda i: (i, 0)),
                  pl.BlockSpec((4, D, D), lambda i: (0, 0, 0)),
                  pl.BlockSpec((4, D), lambda i: (0, 0))],
        out_specs=pl.BlockSpec((blk, D), lambda i: (i, 0)),
        out_shape=jax.ShapeDtypeStruct((n, D), jnp.float32),
    )(x, ws, bs)


EBLK = 2560


def _attn_body(gs_ref, gd_ref, eh_ref, attn_ref, m_ref):
    i = pl.program_id(0)
    a = jnp.sum(gs_ref[...] * eh_ref[...] * gd_ref[...], axis=1, keepdims=True)
    attn_ref[...] = a
    bm = jnp.max(a)

    @pl.when(i == 0)
    def _():
        m_ref[0, 0] = bm

    @pl.when(i > 0)
    def _():
        m_ref[0, 0] = jnp.maximum(m_ref[0, 0], bm)


def _attn(gsrc, gdst, eh):
    return pl.pallas_call(
        _attn_body,
        grid=(NE // EBLK,),
        in_specs=[pl.BlockSpec((EBLK, D), lambda i: (i, 0)),
                  pl.BlockSpec((EBLK, D), lambda i: (i, 0)),
                  pl.BlockSpec((EBLK, D), lambda i: (i, 0))],
        out_specs=[pl.BlockSpec((EBLK, 1), lambda i: (i, 0)),
                   pl.BlockSpec((1, 1), lambda i: (0, 0),
                                memory_space=pltpu.SMEM)],
        out_shape=[jax.ShapeDtypeStruct((NE, 1), jnp.float32),
                   jax.ShapeDtypeStruct((1, 1), jnp.float32)],
    )(gsrc, gdst, eh)


def _pw_body(attn_ref, m_ref, gs_ref, w_ref, p_ref):
    p = jnp.exp(attn_ref[...] - m_ref[0, 0])
    w_ref[...] = gs_ref[...] * p
    p_ref[...] = p


def _pw(attn, m, gsrc):
    return pl.pallas_call(
        _pw_body,
        grid=(NE // EBLK,),
        in_specs=[pl.BlockSpec((EBLK, 1), lambda i: (i, 0)),
                  pl.BlockSpec((1, 1), lambda i: (0, 0),
                               memory_space=pltpu.SMEM),
                  pl.BlockSpec((EBLK, D), lambda i: (i, 0))],
        out_specs=[pl.BlockSpec((EBLK, D), lambda i: (i, 0)),
                   pl.BlockSpec((EBLK, 1), lambda i: (i, 0))],
        out_shape=[jax.ShapeDtypeStruct((NE, D), jnp.float32),
                   jax.ShapeDtypeStruct((NE, 1), jnp.float32)],
    )(attn, m, gsrc)


def _sc_gather(table, idx):
    d = table.shape[1]

    @functools.partial(
        pl.kernel, mesh=_mesh(),
        out_type=jax.ShapeDtypeStruct((idx.shape[0], d), jnp.float32),
        scratch_types=[pltpu.VMEM((PERW,), jnp.int32),
                       pltpu.VMEM((3, CH, d), jnp.float32),
                       pltpu.SemaphoreType.DMA((3,)),
                       pltpu.SemaphoreType.DMA((3,))],
    )
    def k(table_hbm, idx_hbm, out_hbm, idx_all, rows, semg, semw):
        wid = lax.axis_index("s") * NCORE + lax.axis_index("c")
        base = wid * PERW
        pltpu.sync_copy(idx_hbm.at[pl.ds(base, PERW)], idx_all)

        def g_desc(c, s):
            return pltpu.make_async_copy(
                table_hbm.at[idx_all.at[pl.ds(c * CH, CH)]],
                rows.at[s], semg.at[s])

        def w_desc(c, s):
            return pltpu.make_async_copy(
                rows.at[s], out_hbm.at[pl.ds(base + c * CH, CH)], semw.at[s])

        def step(c, s, sp, first):
            g_desc(c, s).wait()
            w_desc(c, s).start()
            nxt = c + 2

            @pl.when(nxt < NFULL)
            def _():
                if not first:
                    w_desc(nxt - 3, sp).wait()
                g_desc(nxt, sp).start()

        g_desc(0, 0).start()
        g_desc(1, 1).start()
        step(0, 0, 2, True)
        step(1, 1, 0, False)
        step(2, 2, 1, False)

        @pl.loop(1, NFULL // 3)
        def _(kk):
            c = kk * 3
            step(c, 0, 2, False)
            step(c + 1, 1, 0, False)
            step(c + 2, 2, 1, False)

        w_desc(NFULL - 3, (NFULL - 3) % 3).wait()
        w_desc(NFULL - 2, (NFULL - 2) % 3).wait()
        w_desc(NFULL - 1, (NFULL - 1) % 3).wait()

        if TAIL:
            off = base + NFULL * CH
            rows_t = rows.at[0].at[pl.ds(0, TAIL)]
            pltpu.async_copy(
                table_hbm.at[idx_all.at[pl.ds(NFULL * CH, TAIL)]],
                rows_t, semg.at[0]).wait()
            pltpu.sync_copy(rows_t, out_hbm.at[pl.ds(off, TAIL)])

    return k(table, idx)


def _sc_scatter(w, p, dstidx, zeros):
    zr0 = 632
    zr_last = NN - 15 * zr0

    @functools.partial(
        pl.kernel, mesh=_mesh(),
        out_type=[jax.ShapeDtypeStruct((NCORE, NN, D), jnp.float32),
                  jax.ShapeDtypeStruct((NWORK, NN), jnp.float32)],
        scratch_types=[pltpu.VMEM((3, SCH), jnp.int32),
                       pltpu.VMEM((3, SCH, D), jnp.float32),
                       pltpu.VMEM((3, SCH), jnp.float32),
                       pltpu.VMEM((NN,), jnp.float32),
                       pltpu.VMEM_SHARED((NN, D), jnp.float32),
                       pltpu.SemaphoreType.DMA((3,)),
                       pltpu.SemaphoreType.DMA((3,)),
                       pltpu.SemaphoreType.DMA((3,)),
                       pltpu.SemaphoreType.DMA((3,))],
        compiler_params=pltpu.CompilerParams(needs_layout_passes=False),
    )
    def k(w_hbm, p_hbm, dst_hbm, z_hbm, out_hbm, s_hbm,
          idx_v, rows_v, p_v, s_tile, acc_sh, semi, semr, semp, sems):
        c = lax.axis_index("c")
        s = lax.axis_index("s")

        def rows_slice(src_ref, dst_ref):
            @pl.when(s < 15)
            def _():
                pltpu.sync_copy(src_ref.at[pl.ds(s * zr0, zr0)],
                                dst_ref.at[pl.ds(s * zr0, zr0)])

            @pl.when(s == 15)
            def _():
                pltpu.sync_copy(src_ref.at[pl.ds(15 * zr0, zr_last)],
                                dst_ref.at[pl.ds(15 * zr0, zr_last)])

        rows_slice(z_hbm, acc_sh)
        zvec = jnp.zeros((16,), jnp.float32)

        @pl.loop(0, NN, step=16)
        def _(i):
            s_tile[pl.ds(i, 16)] = zvec

        plsc.subcore_barrier()

        base = (c * NSUB + s) * PERW

        def i_desc(cc, sl):
            return pltpu.make_async_copy(
                dst_hbm.at[pl.ds(base + cc * SCH, SCH)], idx_v.at[sl],
                semi.at[sl])

        def r_desc(cc, sl):
            return pltpu.make_async_copy(
                w_hbm.at[pl.ds(base + cc * SCH, SCH)], rows_v.at[sl],
                semr.at[sl])

        def p_desc(cc, sl):
            return pltpu.make_async_copy(
                p_hbm.at[pl.ds(base + cc * SCH, SCH)], p_v.at[sl],
                semp.at[sl])

        def s_start(sl):
            pltpu.async_copy(rows_v.at[sl], acc_sh.at[idx_v.at[sl]],
                             sems.at[sl], add=True)

        def s_wait(sl):
            pltpu.make_async_copy(rows_v.at[sl], acc_sh.at[idx_v.at[sl]],
                                  sems.at[sl]).wait()

        def alu(sl):
            for g in range(SCH // 16):
                ii = idx_v[sl, pl.ds(g * 16, 16)]
                pi = p_v[sl, pl.ds(g * 16, 16)]
                plsc.addupdate_scatter(s_tile, [ii], pi)

        def step(cc, sl, slp, first):
            if not first:
                s_wait(sl)
            i_desc(cc, sl).start()
            r_desc(cc, sl).start()
            p_desc(cc, sl).start()
            i_desc(cc - 1, slp).wait()
            r_desc(cc - 1, slp).wait()
            p_desc(cc - 1, slp).wait()
            s_start(slp)
            alu(slp)

        i_desc(0, 0).start()
        r_desc(0, 0).start()
        p_desc(0, 0).start()
        step(1, 1, 0, True)
        step(2, 2, 1, True)

        @pl.loop(1, (SNF - 5) // 3 + 1)
        def _(kk):
            cc = kk * 3
            step(cc, 0, 2, False)
            step(cc + 1, 1, 0, False)
            step(cc + 2, 2, 1, False)

        step(SNF - 2, (SNF - 2) % 3, (SNF - 3) % 3, False)
        step(SNF - 1, (SNF - 1) % 3, (SNF - 2) % 3, False)
        lsl = (SNF - 1) % 3
        i_desc(SNF - 1, lsl).wait()
        r_desc(SNF - 1, lsl).wait()
        p_desc(SNF - 1, lsl).wait()
        s_start(lsl)
        alu(lsl)
        for sl in range(3):
            s_wait(sl)

        plsc.subcore_barrier()
        rows_slice(acc_sh, out_hbm.at[c])
        pltpu.sync_copy(s_tile, s_hbm.at[c * NSUB + s])

    return k(w, p, dstidx, zeros)


NBLK = 2000


def _scomb_body(s_ref, o_ref):
    o_ref[...] = lax.dot_general(s_ref[...], jnp.ones((NWORK, 1), jnp.float32),
                                 (((0,), (0,)), ((), ())),
                                 preferred_element_type=jnp.float32)


def _scomb(s_part):
    return pl.pallas_call(
        _scomb_body,
        out_shape=jax.ShapeDtypeStruct((NN, 1), jnp.float32),
    )(s_part)


def _fin_body(acc_ref, s_ref, nh_ref, nhout_ref, nz_ref):
    a = acc_ref[0] + acc_ref[1]
    s = s_ref[...]
    nz = jnp.where(s > 0.0, a / s, 0.0)
    nz_ref[...] = nz
    nhout_ref[...] = nh_ref[...] + nz


def _finalize(acc, s_col, n_h):
    return pl.pallas_call(
        _fin_body,
        grid=(NN // NBLK,),
        in_specs=[pl.BlockSpec((NCORE, NBLK, D), lambda i: (0, i, 0)),
                  pl.BlockSpec((NBLK, 1), lambda i: (i, 0)),
                  pl.BlockSpec((NBLK, D), lambda i: (i, 0))],
        out_specs=[pl.BlockSpec((NBLK, D), lambda i: (i, 0)),
                   pl.BlockSpec((NBLK, D), lambda i: (i, 0))],
        out_shape=[jax.ShapeDtypeStruct((NN, D), jnp.float32),
                   jax.ShapeDtypeStruct((NN, D), jnp.float32)],
    )(acc, s_col, n_h)


def _eadd_body(eh_ref, gnz_ref, o_ref):
    o_ref[...] = eh_ref[...] + gnz_ref[...]


def _eadd(e_h, gnz):
    return pl.pallas_call(
        _eadd_body,
        grid=(NE // EBLK,),
        in_specs=[pl.BlockSpec((EBLK, D), lambda i: (i, 0)),
                  pl.BlockSpec((EBLK, D), lambda i: (i, 0))],
        out_specs=pl.BlockSpec((EBLK, D), lambda i: (i, 0)),
        out_shape=jax.ShapeDtypeStruct((NE, D), jnp.float32),
    )(e_h, gnz)


def kernel(nh, eh, edge_index, nf_params, ef_params):
    src, dst = edge_index[0], edge_index[1]
    wn = jnp.stack([w for w, _ in nf_params])
    bn = jnp.stack([b for _, b in nf_params])
    we = jnp.stack([w for w, _ in ef_params])
    be = jnp.stack([b for _, b in ef_params])

    n_h = _mlp(nh, wn, bn, blk=2000)
    e_h = _mlp(eh, we, be, blk=EBLK)

    gsrc = _sc_gather(n_h, src)
    gdst = _sc_gather(n_h, dst)

    attn, m = _attn(gsrc, gdst, e_h)
    w128, p = _pw(attn, m, gsrc)

    zeros = jnp.zeros((NN, D), jnp.float32)
    acc, s_part = _sc_scatter(w128, p.reshape(NE), dst, zeros)

    nh_out, nz = _finalize(acc, _scomb(s_part), n_h)

    gnz = _sc_gather(nz, src)
    eh_out = _eadd(e_h, gnz)
    return (nh_out, eh_out)


# speedup vs baseline: 7.0075x; 1.0147x over previous
import functools

import jax
import jax.numpy as jnp
from jax import lax
from jax.experimental import pallas as pl
from jax.experimental.pallas import tpu as pltpu
from jax.experimental.pallas import tpu_sc as plsc

NN = 10000
NE = 320000
D = 128
NCORE = 2
NSUB = 16
NWORK = NCORE * NSUB
K = 2
H = NE // K
CH = 128
SCH = 80
EBLK = 3200


def _mesh():
    return plsc.VectorSubcoreMesh(core_axis_name="c", subcore_axis_name="s",
                                  num_cores=NCORE, num_subcores=NSUB)


def _mlp_body(x_ref, w_ref, b_ref, o_ref):
    h = x_ref[...]
    for l in range(4):
        h = lax.dot_general(h, w_ref[l], (((1,), (0,)), ((), ())),
                            preferred_element_type=jnp.float32)
        h = jnp.tanh(h + b_ref[l][None, :])
    o_ref[...] = h


def _mlp(x, ws, bs, blk):
    n = x.shape[0]
    return pl.pallas_call(
        _mlp_body,
        grid=(n // blk,),
        in_specs=[pl.BlockSpec((blk, D), lambda i: (i, 0)),
                  pl.BlockSpec((4, D, D), lambda i: (0, 0, 0)),
                  pl.BlockSpec((4, D), lambda i: (0, 0))],
        out_specs=pl.BlockSpec((blk, D), lambda i: (i, 0)),
        out_shape=jax.ShapeDtypeStruct((n, D), jnp.float32),
    )(x, ws, bs)


def _attn_body(gs_ref, gd_ref, eh_ref, attn_ref, m_ref):
    i = pl.program_id(0)
    a = jnp.sum(gs_ref[...] * eh_ref[...] * gd_ref[...], axis=1, keepdims=True)
    attn_ref[...] = a
    bm = jnp.max(a)

    @pl.when(i == 0)
    def _():
        m_ref[0, 0] = bm

    @pl.when(i > 0)
    def _():
        m_ref[0, 0] = jnp.maximum(m_ref[0, 0], bm)


def _attn(gsrc, gdst, eh_full, kk):
    off = kk * (H // EBLK)
    return pl.pallas_call(
        _attn_body,
        grid=(H // EBLK,),
        in_specs=[pl.BlockSpec((EBLK, D), lambda i: (i, 0)),
                  pl.BlockSpec((EBLK, D), lambda i: (i, 0)),
                  pl.BlockSpec((EBLK, D), lambda i, o=off: (i + o, 0))],
        out_specs=[pl.BlockSpec((EBLK, 1), lambda i: (i, 0)),
                   pl.BlockSpec((1, 1), lambda i: (0, 0),
                                memory_space=pltpu.SMEM)],
        out_shape=[jax.ShapeDtypeStruct((H, 1), jnp.float32),
                   jax.ShapeDtypeStruct((1, 1), jnp.float32)],
    )(gsrc, gdst, eh_full)


def _pw_body(attn_ref, m_ref, gs_ref, w_ref, p_ref):
    p = jnp.exp(attn_ref[...] - m_ref[0, 0])
    w_ref[...] = gs_ref[...] * p
    p_ref[...] = p


def _pw(attn, m, gsrc):
    return pl.pallas_call(
        _pw_body,
        grid=(H // EBLK,),
        in_specs=[pl.BlockSpec((EBLK, 1), lambda i: (i, 0)),
                  pl.BlockSpec((1, 1), lambda i: (0, 0),
                               memory_space=pltpu.SMEM),
                  pl.BlockSpec((EBLK, D), lambda i: (i, 0))],
        out_specs=[pl.BlockSpec((EBLK, D), lambda i: (i, 0)),
                   pl.BlockSpec((EBLK, 1), lambda i: (i, 0))],
        out_shape=[jax.ShapeDtypeStruct((H, D), jnp.float32),
                   jax.ShapeDtypeStruct((H, 1), jnp.float32)],
    )(attn, m, gsrc)


def _sc_gather(table, idx):
    d = table.shape[1]
    perw = idx.shape[0] // NWORK
    nfull = perw // CH
    tail = perw - nfull * CH
    assert tail % 8 == 0 and perw % 8 == 0 and nfull >= 5

    @functools.partial(
        pl.kernel, mesh=_mesh(),
        out_type=jax.ShapeDtypeStruct((idx.shape[0], d), jnp.float32),
        scratch_types=[pltpu.VMEM((perw,), jnp.int32),
                       pltpu.VMEM((3, CH, d), jnp.float32),
                       pltpu.SemaphoreType.DMA((3,)),
                       pltpu.SemaphoreType.DMA((3,))],
    )
    def k(table_hbm, idx_hbm, out_hbm, idx_all, rows, semg, semw):
        wid = lax.axis_index("s") * NCORE + lax.axis_index("c")
        base = wid * perw
        pltpu.sync_copy(idx_hbm.at[pl.ds(base, perw)], idx_all)

        def g_desc(c, s):
            return pltpu.make_async_copy(
                table_hbm.at[idx_all.at[pl.ds(c * CH, CH)]],
                rows.at[s], semg.at[s])

        def w_desc(c, s):
            return pltpu.make_async_copy(
                rows.at[s], out_hbm.at[pl.ds(base + c * CH, CH)], semw.at[s])

        def step(c, s, sp, first):
            g_desc(c, s).wait()
            w_desc(c, s).start()
            nxt = c + 2

            @pl.when(nxt < nfull)
            def _():
                if not first:
                    w_desc(nxt - 3, sp).wait()
                g_desc(nxt, sp).start()

        g_desc(0, 0).start()
        g_desc(1, 1).start()
        step(0, 0, 2, True)
        step(1, 1, 0, False)
        step(2, 2, 1, False)

        @pl.loop(1, nfull // 3)
        def _(kk):
            c = kk * 3
            step(c, 0, 2, False)
            step(c + 1, 1, 0, False)
            step(c + 2, 2, 1, False)

        for c in range(3 * (nfull // 3), nfull):
            step(c, c % 3, (c + 2) % 3, False)

        w_desc(nfull - 3, (nfull - 3) % 3).wait()
        w_desc(nfull - 2, (nfull - 2) % 3).wait()
        w_desc(nfull - 1, (nfull - 1) % 3).wait()

        if tail:
            off = base + nfull * CH
            rows_t = rows.at[0].at[pl.ds(0, tail)]
            pltpu.async_copy(
                table_hbm.at[idx_all.at[pl.ds(nfull * CH, tail)]],
                rows_t, semg.at[0]).wait()
            pltpu.sync_copy(rows_t, out_hbm.at[pl.ds(off, tail)])

    return k(table, idx)


def _sc_scatter(w, p, dstidx, zeros):
    perw = dstidx.shape[0] // NWORK
    snf = perw // SCH
    stail = perw - snf * SCH
    assert stail % 8 == 0 and snf >= 5
    zr0 = 632
    zr_last = NN - 15 * zr0

    @functools.partial(
        pl.kernel, mesh=_mesh(),
        out_type=[jax.ShapeDtypeStruct((NCORE, NN, D), jnp.float32),
                  jax.ShapeDtypeStruct((NWORK, NN), jnp.float32)],
        scratch_types=[pltpu.VMEM((3, SCH), jnp.int32),
                       pltpu.VMEM((max(stail, 8),), jnp.int32),
                       pltpu.VMEM((3, SCH, D), jnp.float32),
                       pltpu.VMEM((3, SCH), jnp.float32),
                       pltpu.VMEM((NN,), jnp.float32),
                       pltpu.VMEM_SHARED((NN, D), jnp.float32),
                       pltpu.SemaphoreType.DMA((3,)),
                       pltpu.SemaphoreType.DMA((3,)),
                       pltpu.SemaphoreType.DMA((3,)),
                       pltpu.SemaphoreType.DMA((3,))],
        compiler_params=pltpu.CompilerParams(needs_layout_passes=False),
    )
    def k(w_hbm, p_hbm, dst_hbm, z_hbm, out_hbm, s_hbm,
          idx_v, idx_t, rows_v, p_v, s_tile, acc_sh, semi, semr, semp, sems):
        c = lax.axis_index("c")
        s = lax.axis_index("s")

        def rows_slice(src_ref, dst_ref):
            @pl.when(s < 15)
            def _():
                pltpu.sync_copy(src_ref.at[pl.ds(s * zr0, zr0)],
                                dst_ref.at[pl.ds(s * zr0, zr0)])

            @pl.when(s == 15)
            def _():
                pltpu.sync_copy(src_ref.at[pl.ds(15 * zr0, zr_last)],
                                dst_ref.at[pl.ds(15 * zr0, zr_last)])

        rows_slice(z_hbm, acc_sh)
        zvec = jnp.zeros((16,), jnp.float32)

        @pl.loop(0, NN, step=16)
        def _(i):
            s_tile[pl.ds(i, 16)] = zvec

        plsc.subcore_barrier()

        base = (c * NSUB + s) * perw

        def i_desc(cc, sl):
            return pltpu.make_async_copy(
                dst_hbm.at[pl.ds(base + cc * SCH, SCH)], idx_v.at[sl],
                semi.at[sl])

        def r_desc(cc, sl):
            return pltpu.make_async_copy(
                w_hbm.at[pl.ds(base + cc * SCH, SCH)], rows_v.at[sl],
                semr.at[sl])

        def p_desc(cc, sl):
            return pltpu.make_async_copy(
                p_hbm.at[pl.ds(base + cc * SCH, SCH)], p_v.at[sl],
                semp.at[sl])

        def s_start(sl):
            pltpu.async_copy(rows_v.at[sl], acc_sh.at[idx_v.at[sl]],
                             sems.at[sl], add=True)

        def s_wait(sl):
            pltpu.make_async_copy(rows_v.at[sl], acc_sh.at[idx_v.at[sl]],
                                  sems.at[sl]).wait()

        def alu(sl):
            for g in range(SCH // 16):
                ii = idx_v[sl, pl.ds(g * 16, 16)]
                pi = p_v[sl, pl.ds(g * 16, 16)]
                plsc.addupdate_scatter(s_tile, [ii], pi)

        def step(cc, sl, slp, first):
            if not first:
                s_wait(sl)
            i_desc(cc, sl).start()
            r_desc(cc, sl).start()
            p_desc(cc, sl).start()
            i_desc(cc - 1, slp).wait()
            r_desc(cc - 1, slp).wait()
            p_desc(cc - 1, slp).wait()
            s_start(slp)
            alu(slp)

        i_desc(0, 0).start()
        r_desc(0, 0).start()
        p_desc(0, 0).start()
        step(1, 1, 0, True)
        step(2, 2, 1, True)

        @pl.loop(1, (snf - 5) // 3 + 1)
        def _(kk):
            cc = kk * 3
            step(cc, 0, 2, False)
            step(cc + 1, 1, 0, False)
            step(cc + 2, 2, 1, False)

        for cc in range(3 * ((snf - 5) // 3 + 1), snf):
            step(cc, cc % 3, (cc - 1) % 3, False)

        lsl = (snf - 1) % 3
        i_desc(snf - 1, lsl).wait()
        r_desc(snf - 1, lsl).wait()
        p_desc(snf - 1, lsl).wait()
        s_start(lsl)
        alu(lsl)
        for sl in range(3):
            s_wait(sl)

        if stail:
            off = base + snf * SCH
            rows_t = rows_v.at[0].at[pl.ds(0, stail)]
            pt = p_v.at[0].at[pl.ds(0, stail)]
            pltpu.sync_copy(dst_hbm.at[pl.ds(off, stail)], idx_t)
            pltpu.sync_copy(w_hbm.at[pl.ds(off, stail)], rows_t)
            pltpu.sync_copy(p_hbm.at[pl.ds(off, stail)], pt)
            pltpu.sync_copy(rows_t, acc_sh.at[idx_t], add=True)
            lane = lax.iota(jnp.int32, 16)
            for g in range(0, stail, 16):
                rem = stail - g
                if rem >= 16:
                    ii = idx_t[pl.ds(g, 16)]
                    pi = p_v[0, pl.ds(g, 16)]
                    plsc.addupdate_scatter(s_tile, [ii], pi)
                else:
                    ii = idx_t[pl.ds(stail - 16, 16)]
                    pi = p_v[0, pl.ds(stail - 16, 16)]
                    plsc.addupdate_scatter(s_tile, [ii], pi,
                                           mask=lane >= (16 - rem))

        plsc.subcore_barrier()
        rows_slice(acc_sh, out_hbm.at[c])
        pltpu.sync_copy(s_tile, s_hbm.at[c * NSUB + s])

    return k(w, p, dstidx, zeros)


NBLK = 2000


def _scomb_body(s1_ref, s2_ref, sc_ref, o_ref):
    ones = jnp.ones((NWORK, 1), jnp.float32)
    dn = (((0,), (0,)), ((), ()))
    o_ref[...] = (
        sc_ref[0, 0] * lax.dot_general(s1_ref[...], ones, dn,
                                       preferred_element_type=jnp.float32)
        + sc_ref[0, 1] * lax.dot_general(s2_ref[...], ones, dn,
                                         preferred_element_type=jnp.float32))


def _scomb(s1, s2, scales):
    return pl.pallas_call(
        _scomb_body,
        in_specs=[pl.BlockSpec((NWORK, NN), lambda: (0, 0)),
                  pl.BlockSpec((NWORK, NN), lambda: (0, 0)),
                  pl.BlockSpec((1, 2), lambda: (0, 0),
                               memory_space=pltpu.SMEM)],
        out_specs=pl.BlockSpec((NN, 1), lambda: (0, 0)),
        out_shape=jax.ShapeDtypeStruct((NN, 1), jnp.float32),
    )(s1, s2, scales)


def _fin_body(a1_ref, a2_ref, sc_ref, s_ref, nh_ref, nhout_ref, nz_ref):
    a = (sc_ref[0, 0] * (a1_ref[0] + a1_ref[1])
         + sc_ref[0, 1] * (a2_ref[0] + a2_ref[1]))
    s = s_ref[...]
    nz = jnp.where(s > 0.0, a / s, 0.0)
    nz_ref[...] = nz
    nhout_ref[...] = nh_ref[...] + nz


def _finalize(acc1, acc2, scales, s_col, n_h):
    return pl.pallas_call(
        _fin_body,
        grid=(NN // NBLK,),
        in_specs=[pl.BlockSpec((NCORE, NBLK, D), lambda i: (0, i, 0)),
                  pl.BlockSpec((NCORE, NBLK, D), lambda i: (0, i, 0)),
                  pl.BlockSpec((1, 2), lambda i: (0, 0),
                               memory_space=pltpu.SMEM),
                  pl.BlockSpec((NBLK, 1), lambda i: (i, 0)),
                  pl.BlockSpec((NBLK, D), lambda i: (i, 0))],
        out_specs=[pl.BlockSpec((NBLK, D), lambda i: (i, 0)),
                   pl.BlockSpec((NBLK, D), lambda i: (i, 0))],
        out_shape=[jax.ShapeDtypeStruct((NN, D), jnp.float32),
                   jax.ShapeDtypeStruct((NN, D), jnp.float32)],
    )(acc1, acc2, scales, s_col, n_h)


def _eadd_body(eh_ref, gnz_ref, o_ref):
    o_ref[...] = eh_ref[...] + gnz_ref[...]


def _eadd(e_h, gnz):
    return pl.pallas_call(
        _eadd_body,
        grid=(NE // EBLK,),
        in_specs=[pl.BlockSpec((EBLK, D), lambda i: (i, 0)),
                  pl.BlockSpec((EBLK, D), lambda i: (i, 0))],
        out_specs=pl.BlockSpec((EBLK, D), lambda i: (i, 0)),
        out_shape=jax.ShapeDtypeStruct((NE, D), jnp.float32),
    )(e_h, gnz)


def kernel(nh, eh, edge_index, nf_params, ef_params):
    src, dst = edge_index[0], edge_index[1]
    wn = jnp.stack([w for w, _ in nf_params])
    bn = jnp.stack([b for _, b in nf_params])
    we = jnp.stack([w for w, _ in ef_params])
    be = jnp.stack([b for _, b in ef_params])

    n_h = _mlp(nh, wn, bn, blk=2000)
    e_h = _mlp(eh, we, be, blk=EBLK)

    zeros = jnp.zeros((NN, D), jnp.float32)
    accs, sparts, ms = [], [], []
    for kk in range(K):
        src_k = lax.slice(src, (kk * H,), ((kk + 1) * H,))
        dst_k = lax.slice(dst, (kk * H,), ((kk + 1) * H,))
        gsrc = _sc_gather(n_h, src_k)
        gdst = _sc_gather(n_h, dst_k)
        attn, m = _attn(gsrc, gdst, e_h, kk)
        w128, pcol = _pw(attn, m, gsrc)
        acc, s_part = _sc_scatter(w128, pcol.reshape(H), dst_k, zeros)
        accs.append(acc)
        sparts.append(s_part)
        ms.append(m)

    mmax = jnp.maximum(ms[0], ms[1])
    scales = jnp.exp(jnp.concatenate([ms[0], ms[1]], axis=1) - mmax)

    s_col = _scomb(sparts[0], sparts[1], scales)
    nh_out, nz = _finalize(accs[0], accs[1], scales, s_col, n_h)

    gnz = _sc_gather(nz, src)
    eh_out = _eadd(e_h, gnz)
    return (nh_out, eh_out)
